# baseline (speedup 1.0000x reference)
import jax
import jax.numpy as jnp
from jax import lax
from jax.experimental import pallas as pl
from jax.experimental.pallas import tpu as pltpu
from jax.experimental.pallas import tpu_sc as plsc

H = 128
N_NODE = 10000
N_PAD = 10368
E_EDGE = 320000
GRP = 128
NTILE = 32
G_E = 80
E_PAD = NTILE * G_E * GRP
E_LBL = 100000
G_L = 26
L_PAD = NTILE * G_L * GRP
ROWS_PER_TILE = N_PAD // 16

_DN = (((1,), (1,)), ((), ()))


def _matT(x, w):
    return lax.dot_general(x, w, _DN, preferred_element_type=jnp.float32)


def _input_body(xu_ref, xm_ref, luW, lub, lmW, lmb, bug, bub, bmg, bmb,
                ou_ref, om_ref):
    def one(x, W, b, g, bb, o_ref):
        y = _matT(x, W[...]) + b[...]
        m = jnp.mean(y, axis=0, keepdims=True)
        v = jnp.mean((y - m) ** 2, axis=0, keepdims=True)
        act = jnp.maximum((y - m) * lax.rsqrt(v + 1e-5) * g[...] + bb[...], 0.0)
        o_ref[...] = jnp.concatenate(
            [act, jnp.zeros((N_PAD - N_NODE, H), jnp.float32)], axis=0)

    one(xu_ref[...], luW, lub, bug, bub, ou_ref)
    one(xm_ref[...], lmW, lmb, bmg, bmb, om_ref)


def _tc_input(x_user, x_movie, lu_W, lu_b, lm_W, lm_b, bn_u_g, bn_u_b, bn_m_g, bn_m_b):
    return pl.pallas_call(
        _input_body,
        out_shape=[jax.ShapeDtypeStruct((N_PAD, H), jnp.float32),
                   jax.ShapeDtypeStruct((N_PAD, H), jnp.float32)],
    )(x_user, x_movie, lu_W, lu_b.reshape(1, H), lm_W, lm_b.reshape(1, H),
      bn_u_g.reshape(1, H), bn_u_b.reshape(1, H), bn_m_g.reshape(1, H), bn_m_b.reshape(1, H))


def _sc_mesh():
    return plsc.VectorSubcoreMesh(core_axis_name="c", subcore_axis_name="s")


def _sum_edges(wid, table_hbm, e_hbm, acc, idx_s, idx_d, msg, sem):
    def step(j, carry):
        pltpu.sync_copy(e_hbm.at[0, wid, j], idx_s)
        pltpu.sync_copy(e_hbm.at[1, wid, j], idx_d)
        pltpu.async_copy(table_hbm.at[idx_s], msg, sem).wait()
        pltpu.sync_copy(msg, acc.at[idx_d], add=True)
        return carry

    lax.fori_loop(0, G_E, step, 0)


def _cnt_edges(wid, e_hbm, acc, ones_v, idx_d):
    def step(j, carry):
        pltpu.sync_copy(e_hbm.at[1, wid, j], idx_d)
        pltpu.sync_copy(ones_v, acc.at[idx_d], add=True)
        return carry

    lax.fori_loop(0, G_E, step, 0)


def _seg_body_l1(xu_hbm, xm_hbm, eR_hbm, eD_hbm, zs_hbm, ones_hbm,
                 os_m, os_u, oc_m, oc_u,
                 idx_s, idx_d, msg, acc_sum, sem):
    cid = lax.axis_index("c")
    sid = lax.axis_index("s")
    wid = cid * 16 + sid
    r0 = sid * ROWS_PER_TILE

    def zero_acc():
        pltpu.sync_copy(zs_hbm.at[pl.ds(r0, ROWS_PER_TILE)],
                        acc_sum.at[pl.ds(r0, ROWS_PER_TILE)])

    def copy_out(o):
        pltpu.sync_copy(acc_sum.at[pl.ds(r0, ROWS_PER_TILE)],
                        o.at[cid, pl.ds(r0, ROWS_PER_TILE)])

    zero_acc()
    plsc.subcore_barrier()
    _sum_edges(wid, xu_hbm, eR_hbm, acc_sum, idx_s, idx_d, msg, sem)
    plsc.subcore_barrier()
    copy_out(os_m)
    plsc.subcore_barrier()
    zero_acc()
    plsc.subcore_barrier()
    _sum_edges(wid, xm_hbm, eD_hbm, acc_sum, idx_s, idx_d, msg, sem)
    plsc.subcore_barrier()
    copy_out(os_u)
    plsc.subcore_barrier()
    zero_acc()
    pltpu.sync_copy(ones_hbm, msg)
    plsc.subcore_barrier()
    _cnt_edges(wid, eR_hbm, acc_sum, msg, idx_d)
    plsc.subcore_barrier()
    copy_out(oc_m)
    plsc.subcore_barrier()
    zero_acc()
    plsc.subcore_barrier()
    _cnt_edges(wid, eD_hbm, acc_sum, msg, idx_d)
    plsc.subcore_barrier()
    copy_out(oc_u)


def _sc_segsum_l1(xu, xm, eR, eD):
    f32 = jnp.float32
    zs = jnp.zeros((N_PAD, H), f32)
    ones = jnp.ones((GRP, H), f32)
    return pl.kernel(
        _seg_body_l1,
        out_type=[jax.ShapeDtypeStruct((2, N_PAD, H), f32)] * 4,
        mesh=_sc_mesh(),
        scratch_types=[
            pltpu.VMEM((GRP,), jnp.int32),
            pltpu.VMEM((GRP,), jnp.int32),
            pltpu.VMEM((GRP, H), f32),
            pltpu.VMEM_SHARED((N_PAD, H), f32),
            pltpu.SemaphoreType.DMA,
        ],
    )(xu, xm, eR, eD, zs, ones)


def _seg_body_plain(xu_hbm, xm_hbm, eR_hbm, eD_hbm, zs_hbm,
                    os_m, os_u,
                    idx_s, idx_d, msg, acc_sum, sem):
    cid = lax.axis_index("c")
    sid = lax.axis_index("s")
    wid = cid * 16 + sid
    r0 = sid * ROWS_PER_TILE

    def zero_acc():
        pltpu.sync_copy(zs_hbm.at[pl.ds(r0, ROWS_PER_TILE)],
                        acc_sum.at[pl.ds(r0, ROWS_PER_TILE)])

    def copy_out(o):
        pltpu.sync_copy(acc_sum.at[pl.ds(r0, ROWS_PER_TILE)],
                        o.at[cid, pl.ds(r0, ROWS_PER_TILE)])

    zero_acc()
    plsc.subcore_barrier()
    _sum_edges(wid, xu_hbm, eR_hbm, acc_sum, idx_s, idx_d, msg, sem)
    plsc.subcore_barrier()
    copy_out(os_m)
    plsc.subcore_barrier()
    zero_acc()
    plsc.subcore_barrier()
    _sum_edges(wid, xm_hbm, eD_hbm, acc_sum, idx_s, idx_d, msg, sem)
    plsc.subcore_barrier()
    copy_out(os_u)


def _sc_segsum(table_m, table_u, eR, eD):
    f32 = jnp.float32
    zs = jnp.zeros((N_PAD, H), f32)
    return pl.kernel(
        _seg_body_plain,
        out_type=[jax.ShapeDtypeStruct((2, N_PAD, H), f32),
                  jax.ShapeDtypeStruct((2, N_PAD, H), f32)],
        mesh=_sc_mesh(),
        scratch_types=[
            pltpu.VMEM((GRP,), jnp.int32),
            pltpu.VMEM((GRP,), jnp.int32),
            pltpu.VMEM((GRP, H), f32),
            pltpu.VMEM_SHARED((N_PAD, H), f32),
            pltpu.SemaphoreType.DMA,
        ],
    )(table_m, table_u, eR, eD, zs)


def _gather_body(u2_hbm, m2_hbm, eli_hbm, fu_hbm, fm_hbm, idx0, idx1, buf, sem):
    cid = lax.axis_index("c")
    sid = lax.axis_index("s")
    wid = cid * 16 + sid
    base = wid * (G_L * GRP)

    def step(j, carry):
        pltpu.sync_copy(eli_hbm.at[0, wid, j], idx0)
        pltpu.sync_copy(eli_hbm.at[1, wid, j], idx1)
        pltpu.async_copy(u2_hbm.at[idx0], buf, sem).wait()
        pltpu.sync_copy(buf, fu_hbm.at[pl.ds(base + j * GRP, GRP)])
        pltpu.async_copy(m2_hbm.at[idx1], buf, sem).wait()
        pltpu.sync_copy(buf, fm_hbm.at[pl.ds(base + j * GRP, GRP)])
        return carry

    lax.fori_loop(0, G_L, step, 0)


def _sc_label_gather(u2, m2, eli):
    f32 = jnp.float32
    return pl.kernel(
        _gather_body,
        out_type=[jax.ShapeDtypeStruct((L_PAD, H), f32),
                  jax.ShapeDtypeStruct((L_PAD, H), f32)],
        mesh=_sc_mesh(),
        scratch_types=[
            pltpu.VMEM((GRP,), jnp.int32),
            pltpu.VMEM((GRP,), jnp.int32),
            pltpu.VMEM((GRP, H), f32),
            pltpu.SemaphoreType.DMA,
        ],
    )(u2, m2, eli)


def _make_sage_body(relu, out_bf16):
    def body(sm0, sm1, cm0, cm1, xdm, Wlm, blm, Wrm,
             su0, su1, cu0, cu1, xdu, Wlu, blu, Wru, om_ref, ou_ref):
        def one(s0, s1, c0, c1, xd, Wl, bl, Wr, o_ref):
            cnt = jnp.maximum(c0[:, 0:1] + c1[:, 0:1], 1.0)
            mean = (s0[...] + s1[...]) / cnt
            r = _matT(mean, Wl[...]) + bl[...] + _matT(xd[...], Wr[...])
            if relu:
                r = jnp.maximum(r, 0.0)
            o_ref[...] = r.astype(jnp.bfloat16) if out_bf16 else r

        one(sm0, sm1, cm0, cm1, xdm, Wlm, blm, Wrm, om_ref)
        one(su0, su1, cu0, cu1, xdu, Wlu, blu, Wru, ou_ref)

    return body


_SAGE_BLK = 2592


def _tc_sage(relu, out_bf16, sm, cm, xdm, Wlm, blm, Wrm, su, cu, xdu, Wlu, blu, Wru):
    row = pl.BlockSpec((_SAGE_BLK, H), lambda i: (i, 0))
    wfull = pl.BlockSpec((H, H), lambda i: (0, 0))
    bfull = pl.BlockSpec((1, H), lambda i: (0, 0))
    grid = (N_PAD // _SAGE_BLK,)
    odt = jnp.bfloat16 if out_bf16 else jnp.float32
    return pl.pallas_call(
        _make_sage_body(relu, out_bf16),
        grid=grid,
        in_specs=[row, row, row, row, row, wfull, bfull, wfull,
                  row, row, row, row, row, wfull, bfull, wfull],
        out_specs=[row, row],
        out_shape=[jax.ShapeDtypeStruct((N_PAD, H), odt),
                   jax.ShapeDtypeStruct((N_PAD, H), odt)],
    )(sm[0], sm[1], cm[0], cm[1], xdm, Wlm, blm.reshape(1, H), Wrm,
      su[0], su[1], cu[0], cu[1], xdu, Wlu, blu.reshape(1, H), Wru)


_DEC_BLK = 2048


def _dec_body(fu, fm, W1a, W1b, b1, W2, b2, W3, b3, o_ref):
    h = jnp.maximum(_matT(fu[...].astype(jnp.bfloat16), W1a[...])
                    + _matT(fm[...].astype(jnp.bfloat16), W1b[...]) + b1[...], 0.0)
    h = jnp.maximum(_matT(h.astype(jnp.bfloat16), W2[...]) + b2[...], 0.0)
    o_ref[...] = _matT(h.astype(jnp.bfloat16), W3[...]) + b3[...]


def _tc_decoder(fu, fm, d_W1, d_b1, d_W2, d_b2, d_W3, d_b3):
    H4, H2 = 4 * H, 2 * H
    bf16 = jnp.bfloat16
    W1a = d_W1[:, :H].astype(bf16)
    W1b = d_W1[:, H:].astype(bf16)
    W2b = d_W2.astype(bf16)
    W3p = jnp.zeros((8, H2), jnp.float32).at[0].set(d_W3[0]).astype(bf16)
    b3p = jnp.broadcast_to(d_b3.reshape(1, 1), (_DEC_BLK, 8))
    row = pl.BlockSpec((_DEC_BLK, H), lambda i: (i, 0))
    grid = (L_PAD // _DEC_BLK,)
    out = pl.pallas_call(
        _dec_body,
        grid=grid,
        in_specs=[row, row,
                  pl.BlockSpec((H4, H), lambda i: (0, 0)),
                  pl.BlockSpec((H4, H), lambda i: (0, 0)),
                  pl.BlockSpec((1, H4), lambda i: (0, 0)),
                  pl.BlockSpec((H2, H4), lambda i: (0, 0)),
                  pl.BlockSpec((1, H2), lambda i: (0, 0)),
                  pl.BlockSpec((8, H2), lambda i: (0, 0)),
                  pl.BlockSpec((_DEC_BLK, 8), lambda i: (0, 0))],
        out_specs=pl.BlockSpec((_DEC_BLK, 8), lambda i: (i, 0)),
        out_shape=jax.ShapeDtypeStruct((L_PAD, 8), jnp.float32),
    )(fu, fm, W1a, W1b, d_b1.reshape(1, H4), W2b, d_b2.reshape(1, H2),
      W3p, b3p)
    return out[:E_LBL, 0]


def _pad_edges(ei):
    pad = E_PAD - E_EDGE
    src = jnp.concatenate([ei[0], jnp.zeros((pad,), jnp.int32)])
    dst = jnp.concatenate([ei[1], jnp.full((pad,), N_NODE, jnp.int32)])
    return jnp.stack([src, dst]).reshape(2, NTILE, G_E, GRP)


def kernel(x_user, x_movie, ei_rates, ei_rated, edge_label_index,
           lu_W, lu_b, lm_W, lm_b, bn_u_g, bn_u_b, bn_m_g, bn_m_b,
           c1r_Wl, c1r_bl, c1r_Wr, c1d_Wl, c1d_bl, c1d_Wr,
           c2r_Wl, c2r_bl, c2r_Wr, c2d_Wl, c2d_bl, c2d_Wr,
           d_W1, d_b1, d_W2, d_b2, d_W3, d_b3):
    eR = _pad_edges(ei_rates)
    eD = _pad_edges(ei_rated)
    lpad = L_PAD - E_LBL
    eli = jnp.concatenate([edge_label_index,
                           jnp.zeros((2, lpad), jnp.int32)], axis=1).reshape(2, NTILE, G_L, GRP)

    xu, xm = _tc_input(x_user, x_movie, lu_W, lu_b, lm_W, lm_b,
                       bn_u_g, bn_u_b, bn_m_g, bn_m_b)

    os_m, os_u, oc_m, oc_u = _sc_segsum_l1(xu, xm, eR, eD)

    m1, u1 = _tc_sage(True, False, os_m, oc_m, xm, c1r_Wl, c1r_bl, c1r_Wr,
                      os_u, oc_u, xu, c1d_Wl, c1d_bl, c1d_Wr)

    os2_m, os2_u = _sc_segsum(u1, m1, eR, eD)

    m2, u2 = _tc_sage(False, False, os2_m, oc_m, m1, c2r_Wl, c2r_bl, c2r_Wr,
                      os2_u, oc_u, u1, c2d_Wl, c2d_bl, c2d_Wr)

    fu, fm = _sc_label_gather(u2, m2, eli)

    return _tc_decoder(fu, fm, d_W1, d_b1, d_W2, d_b2, d_W3, d_b3)


# speedup vs baseline: 1.0534x; 1.0534x over previous
import jax
import jax.numpy as jnp
from jax import lax
from jax.experimental import pallas as pl
from jax.experimental.pallas import tpu as pltpu
from jax.experimental.pallas import tpu_sc as plsc

H = 128
N_NODE = 10000
N_PAD = 10368
E_EDGE = 320000
GRP = 128
NTILE = 32
G_E = 80
E_PAD = NTILE * G_E * GRP
E_LBL = 100000
G_L = 26
L_PAD = NTILE * G_L * GRP
ROWS_PER_TILE = N_PAD // 16

_DN = (((1,), (1,)), ((), ()))


def _matT(x, w):
    return lax.dot_general(x, w, _DN, preferred_element_type=jnp.float32)


def _input_body(xu_ref, xm_ref, luW, lub, lmW, lmb, bug, bub, bmg, bmb,
                ou_ref, om_ref):
    def one(x, W, b, g, bb, o_ref):
        y = _matT(x, W[...]) + b[...]
        m = jnp.mean(y, axis=0, keepdims=True)
        v = jnp.mean((y - m) ** 2, axis=0, keepdims=True)
        act = jnp.maximum((y - m) * lax.rsqrt(v + 1e-5) * g[...] + bb[...], 0.0)
        o_ref[...] = jnp.concatenate(
            [act, jnp.zeros((N_PAD - N_NODE, H), jnp.float32)], axis=0)

    one(xu_ref[...], luW, lub, bug, bub, ou_ref)
    one(xm_ref[...], lmW, lmb, bmg, bmb, om_ref)


def _tc_input(x_user, x_movie, lu_W, lu_b, lm_W, lm_b, bn_u_g, bn_u_b, bn_m_g, bn_m_b):
    return pl.pallas_call(
        _input_body,
        out_shape=[jax.ShapeDtypeStruct((N_PAD, H), jnp.float32),
                   jax.ShapeDtypeStruct((N_PAD, H), jnp.float32)],
    )(x_user, x_movie, lu_W, lu_b.reshape(1, H), lm_W, lm_b.reshape(1, H),
      bn_u_g.reshape(1, H), bn_u_b.reshape(1, H), bn_m_g.reshape(1, H), bn_m_b.reshape(1, H))


def _sc_mesh():
    return plsc.VectorSubcoreMesh(core_axis_name="c", subcore_axis_name="s")


def _sum_edges(wid, table_hbm, e_hbm, acc, idx_s, idx_d, msg, sem):
    def step(j, carry):
        pltpu.sync_copy(e_hbm.at[0, wid, j], idx_s)
        pltpu.sync_copy(e_hbm.at[1, wid, j], idx_d)
        pltpu.async_copy(table_hbm.at[idx_s], msg, sem).wait()
        pltpu.sync_copy(msg, acc.at[idx_d], add=True)
        return carry

    lax.fori_loop(0, G_E, step, 0)


def _cnt_edges(wid, e_hbm, acc, ones_v, idx_d):
    def step(j, carry):
        pltpu.sync_copy(e_hbm.at[1, wid, j], idx_d)
        pltpu.sync_copy(ones_v, acc.at[idx_d], add=True)
        return carry

    lax.fori_loop(0, G_E, step, 0)


def _seg_body_l1(xu_hbm, xm_hbm, eR_hbm, eD_hbm, zs_hbm, ones_hbm,
                 os_m, os_u, oc_m, oc_u,
                 idx_s, idx_d, msg, acc_sum, sem):
    cid = lax.axis_index("c")
    sid = lax.axis_index("s")
    wid = cid * 16 + sid
    r0 = sid * ROWS_PER_TILE

    def zero_acc():
        pltpu.sync_copy(zs_hbm.at[pl.ds(r0, ROWS_PER_TILE)],
                        acc_sum.at[pl.ds(r0, ROWS_PER_TILE)])

    def copy_out(o):
        pltpu.sync_copy(acc_sum.at[pl.ds(r0, ROWS_PER_TILE)],
                        o.at[cid, pl.ds(r0, ROWS_PER_TILE)])

    zero_acc()
    plsc.subcore_barrier()
    _sum_edges(wid, xu_hbm, eR_hbm, acc_sum, idx_s, idx_d, msg, sem)
    plsc.subcore_barrier()
    copy_out(os_m)
    plsc.subcore_barrier()
    zero_acc()
    plsc.subcore_barrier()
    _sum_edges(wid, xm_hbm, eD_hbm, acc_sum, idx_s, idx_d, msg, sem)
    plsc.subcore_barrier()
    copy_out(os_u)
    plsc.subcore_barrier()
    zero_acc()
    pltpu.sync_copy(ones_hbm, msg)
    plsc.subcore_barrier()
    _cnt_edges(wid, eR_hbm, acc_sum, msg, idx_d)
    plsc.subcore_barrier()
    copy_out(oc_m)
    plsc.subcore_barrier()
    zero_acc()
    plsc.subcore_barrier()
    _cnt_edges(wid, eD_hbm, acc_sum, msg, idx_d)
    plsc.subcore_barrier()
    copy_out(oc_u)


def _sc_segsum_l1(xu, xm, eR, eD):
    f32 = jnp.float32
    zs = jnp.zeros((N_PAD, H), f32)
    ones = jnp.ones((GRP, H), f32)
    return pl.kernel(
        _seg_body_l1,
        out_type=[jax.ShapeDtypeStruct((2, N_PAD, H), f32)] * 4,
        mesh=_sc_mesh(),
        scratch_types=[
            pltpu.VMEM((GRP,), jnp.int32),
            pltpu.VMEM((GRP,), jnp.int32),
            pltpu.VMEM((GRP, H), f32),
            pltpu.VMEM_SHARED((N_PAD, H), f32),
            pltpu.SemaphoreType.DMA,
        ],
    )(xu, xm, eR, eD, zs, ones)


def _seg_body_plain(xu_hbm, xm_hbm, eR_hbm, eD_hbm, zs_hbm,
                    os_m, os_u,
                    idx_s, idx_d, msg, acc_sum, sem):
    cid = lax.axis_index("c")
    sid = lax.axis_index("s")
    wid = cid * 16 + sid
    r0 = sid * ROWS_PER_TILE

    def zero_acc():
        pltpu.sync_copy(zs_hbm.at[pl.ds(r0, ROWS_PER_TILE)],
                        acc_sum.at[pl.ds(r0, ROWS_PER_TILE)])

    def copy_out(o):
        pltpu.sync_copy(acc_sum.at[pl.ds(r0, ROWS_PER_TILE)],
                        o.at[cid, pl.ds(r0, ROWS_PER_TILE)])

    zero_acc()
    plsc.subcore_barrier()
    _sum_edges(wid, xu_hbm, eR_hbm, acc_sum, idx_s, idx_d, msg, sem)
    plsc.subcore_barrier()
    copy_out(os_m)
    plsc.subcore_barrier()
    zero_acc()
    plsc.subcore_barrier()
    _sum_edges(wid, xm_hbm, eD_hbm, acc_sum, idx_s, idx_d, msg, sem)
    plsc.subcore_barrier()
    copy_out(os_u)


def _sc_segsum(table_m, table_u, eR, eD):
    f32 = jnp.float32
    zs = jnp.zeros((N_PAD, H), f32)
    return pl.kernel(
        _seg_body_plain,
        out_type=[jax.ShapeDtypeStruct((2, N_PAD, H), f32),
                  jax.ShapeDtypeStruct((2, N_PAD, H), f32)],
        mesh=_sc_mesh(),
        scratch_types=[
            pltpu.VMEM((GRP,), jnp.int32),
            pltpu.VMEM((GRP,), jnp.int32),
            pltpu.VMEM((GRP, H), f32),
            pltpu.VMEM_SHARED((N_PAD, H), f32),
            pltpu.SemaphoreType.DMA,
        ],
    )(table_m, table_u, eR, eD, zs)


def _gather_body(u2_hbm, m2_hbm, eli_hbm, fu_hbm, fm_hbm, idx0, idx1, buf, sem):
    cid = lax.axis_index("c")
    sid = lax.axis_index("s")
    wid = cid * 16 + sid
    base = wid * (G_L * GRP)

    def step(j, carry):
        pltpu.sync_copy(eli_hbm.at[0, wid, j], idx0)
        pltpu.sync_copy(eli_hbm.at[1, wid, j], idx1)
        pltpu.async_copy(u2_hbm.at[idx0], buf, sem).wait()
        pltpu.sync_copy(buf, fu_hbm.at[pl.ds(base + j * GRP, GRP)])
        pltpu.async_copy(m2_hbm.at[idx1], buf, sem).wait()
        pltpu.sync_copy(buf, fm_hbm.at[pl.ds(base + j * GRP, GRP)])
        return carry

    lax.fori_loop(0, G_L, step, 0)


def _sc_label_gather(u2, m2, eli):
    f32 = jnp.float32
    return pl.kernel(
        _gather_body,
        out_type=[jax.ShapeDtypeStruct((L_PAD, H), f32),
                  jax.ShapeDtypeStruct((L_PAD, H), f32)],
        mesh=_sc_mesh(),
        scratch_types=[
            pltpu.VMEM((GRP,), jnp.int32),
            pltpu.VMEM((GRP,), jnp.int32),
            pltpu.VMEM((GRP, H), f32),
            pltpu.SemaphoreType.DMA,
        ],
    )(u2, m2, eli)


def _make_sage_body(relu, out_bf16):
    def body(sm0, sm1, cm0, cm1, xdm, Wlm, blm, Wrm,
             su0, su1, cu0, cu1, xdu, Wlu, blu, Wru, om_ref, ou_ref):
        def one(s0, s1, c0, c1, xd, Wl, bl, Wr, o_ref):
            cnt = jnp.maximum(c0[:, 0:1] + c1[:, 0:1], 1.0)
            mean = (s0[...] + s1[...]) / cnt
            r = _matT(mean, Wl[...]) + bl[...] + _matT(xd[...], Wr[...])
            if relu:
                r = jnp.maximum(r, 0.0)
            o_ref[...] = r.astype(jnp.bfloat16) if out_bf16 else r

        one(sm0, sm1, cm0, cm1, xdm, Wlm, blm, Wrm, om_ref)
        one(su0, su1, cu0, cu1, xdu, Wlu, blu, Wru, ou_ref)

    return body


_SAGE_BLK = 2592


def _tc_sage(relu, out_bf16, sm, cm, xdm, Wlm, blm, Wrm, su, cu, xdu, Wlu, blu, Wru):
    row = pl.BlockSpec((_SAGE_BLK, H), lambda i: (i, 0))
    wfull = pl.BlockSpec((H, H), lambda i: (0, 0))
    bfull = pl.BlockSpec((1, H), lambda i: (0, 0))
    grid = (N_PAD // _SAGE_BLK,)
    odt = jnp.bfloat16 if out_bf16 else jnp.float32
    return pl.pallas_call(
        _make_sage_body(relu, out_bf16),
        grid=grid,
        in_specs=[row, row, row, row, row, wfull, bfull, wfull,
                  row, row, row, row, row, wfull, bfull, wfull],
        out_specs=[row, row],
        out_shape=[jax.ShapeDtypeStruct((N_PAD, H), odt),
                   jax.ShapeDtypeStruct((N_PAD, H), odt)],
    )(sm[0], sm[1], cm[0], cm[1], xdm, Wlm, blm.reshape(1, H), Wrm,
      su[0], su[1], cu[0], cu[1], xdu, Wlu, blu.reshape(1, H), Wru)


_DEC_BLK = 2048


def _dec_body(fu, fm, W1a, W1b, b1, W2, b2, W3, b3, o_ref):
    h = jnp.maximum(_matT(fu[...].astype(jnp.bfloat16), W1a[...])
                    + _matT(fm[...].astype(jnp.bfloat16), W1b[...]) + b1[...], 0.0)
    h = jnp.maximum(_matT(h.astype(jnp.bfloat16), W2[...]) + b2[...], 0.0)
    o_ref[...] = _matT(h.astype(jnp.bfloat16), W3[...]) + b3[...]


def _tc_decoder(fu, fm, d_W1, d_b1, d_W2, d_b2, d_W3, d_b3):
    H4, H2 = 4 * H, 2 * H
    bf16 = jnp.bfloat16
    W1a = d_W1[:, :H].astype(bf16)
    W1b = d_W1[:, H:].astype(bf16)
    W2b = d_W2.astype(bf16)
    W3p = jnp.zeros((8, H2), jnp.float32).at[0].set(d_W3[0]).astype(bf16)
    b3p = jnp.broadcast_to(d_b3.reshape(1, 1), (_DEC_BLK, 8))
    row = pl.BlockSpec((_DEC_BLK, H), lambda i: (i, 0))
    grid = (L_PAD // _DEC_BLK,)
    out = pl.pallas_call(
        _dec_body,
        grid=grid,
        in_specs=[row, row,
                  pl.BlockSpec((H4, H), lambda i: (0, 0)),
                  pl.BlockSpec((H4, H), lambda i: (0, 0)),
                  pl.BlockSpec((1, H4), lambda i: (0, 0)),
                  pl.BlockSpec((H2, H4), lambda i: (0, 0)),
                  pl.BlockSpec((1, H2), lambda i: (0, 0)),
                  pl.BlockSpec((8, H2), lambda i: (0, 0)),
                  pl.BlockSpec((_DEC_BLK, 8), lambda i: (0, 0))],
        out_specs=pl.BlockSpec((_DEC_BLK, 8), lambda i: (i, 0)),
        out_shape=jax.ShapeDtypeStruct((L_PAD, 8), jnp.float32),
    )(fu, fm, W1a, W1b, d_b1.reshape(1, H4), W2b, d_b2.reshape(1, H2),
      W3p, b3p)
    return out[:E_LBL, 0]


def _pad_edges(ei):
    pad = E_PAD - E_EDGE
    trash = N_NODE + jnp.arange(pad, dtype=jnp.int32) % (N_PAD - N_NODE)
    src = jnp.concatenate([ei[0], jnp.zeros((pad,), jnp.int32)])
    dst = jnp.concatenate([ei[1], trash])
    return jnp.stack([src, dst]).reshape(2, NTILE, G_E, GRP)


def kernel(x_user, x_movie, ei_rates, ei_rated, edge_label_index,
           lu_W, lu_b, lm_W, lm_b, bn_u_g, bn_u_b, bn_m_g, bn_m_b,
           c1r_Wl, c1r_bl, c1r_Wr, c1d_Wl, c1d_bl, c1d_Wr,
           c2r_Wl, c2r_bl, c2r_Wr, c2d_Wl, c2d_bl, c2d_Wr,
           d_W1, d_b1, d_W2, d_b2, d_W3, d_b3):
    eR = _pad_edges(ei_rates)
    eD = _pad_edges(ei_rated)
    lpad = L_PAD - E_LBL
    eli = jnp.concatenate([edge_label_index,
                           jnp.zeros((2, lpad), jnp.int32)], axis=1).reshape(2, NTILE, G_L, GRP)

    xu, xm = _tc_input(x_user, x_movie, lu_W, lu_b, lm_W, lm_b,
                       bn_u_g, bn_u_b, bn_m_g, bn_m_b)

    os_m, os_u, oc_m, oc_u = _sc_segsum_l1(xu, xm, eR, eD)

    m1, u1 = _tc_sage(True, False, os_m, oc_m, xm, c1r_Wl, c1r_bl, c1r_Wr,
                      os_u, oc_u, xu, c1d_Wl, c1d_bl, c1d_Wr)

    os2_m, os2_u = _sc_segsum(u1, m1, eR, eD)

    m2, u2 = _tc_sage(False, False, os2_m, oc_m, m1, c2r_Wl, c2r_bl, c2r_Wr,
                      os2_u, oc_u, u1, c2d_Wl, c2d_bl, c2d_Wr)

    fu, fm = _sc_label_gather(u2, m2, eli)

    return _tc_decoder(fu, fm, d_W1, d_b1, d_W2, d_b2, d_W3, d_b3)


# speedup vs baseline: 1.4432x; 1.3699x over previous
import jax
import jax.numpy as jnp
from jax import lax
from jax.experimental import pallas as pl
from jax.experimental.pallas import tpu as pltpu
from jax.experimental.pallas import tpu_sc as plsc

H = 128
N_NODE = 10000
N_PAD = 10368
E_EDGE = 320000
GRP = 128
NTILE = 32
G_E = 79
E_PAD = NTILE * G_E * GRP
E_LBL = 100000
G_L = 25
L_PAD = NTILE * G_L * GRP
ROWS_PER_TILE = N_PAD // 16

_DN = (((1,), (1,)), ((), ()))


def _matT(x, w):
    return lax.dot_general(x, w, _DN, preferred_element_type=jnp.float32)


def _input_body(xu_ref, xm_ref, luW, lub, lmW, lmb, bug, bub, bmg, bmb,
                ou_ref, om_ref):
    def one(x, W, b, g, bb, o_ref):
        y = _matT(x, W[...]) + b[...]
        m = jnp.mean(y, axis=0, keepdims=True)
        v = jnp.mean((y - m) ** 2, axis=0, keepdims=True)
        act = jnp.maximum((y - m) * lax.rsqrt(v + 1e-5) * g[...] + bb[...], 0.0)
        o_ref[...] = jnp.concatenate(
            [act, jnp.zeros((N_PAD - N_NODE, H), jnp.float32)], axis=0)

    one(xu_ref[...], luW, lub, bug, bub, ou_ref)
    one(xm_ref[...], lmW, lmb, bmg, bmb, om_ref)


def _tc_input(x_user, x_movie, lu_W, lu_b, lm_W, lm_b, bn_u_g, bn_u_b, bn_m_g, bn_m_b):
    return pl.pallas_call(
        _input_body,
        out_shape=[jax.ShapeDtypeStruct((N_PAD, H), jnp.float32),
                   jax.ShapeDtypeStruct((N_PAD, H), jnp.float32)],
    )(x_user, x_movie, lu_W, lu_b.reshape(1, H), lm_W, lm_b.reshape(1, H),
      bn_u_g.reshape(1, H), bn_u_b.reshape(1, H), bn_m_g.reshape(1, H), bn_m_b.reshape(1, H))


def _sc_mesh():
    return plsc.VectorSubcoreMesh(core_axis_name="c", subcore_axis_name="s")


def _sum_edges(wid, table_hbm, e_hbm, acc, idx_s, idx_d, msg, sem):
    def step(j, carry):
        pltpu.sync_copy(e_hbm.at[0, wid, j], idx_s)
        pltpu.sync_copy(e_hbm.at[1, wid, j], idx_d)
        pltpu.async_copy(table_hbm.at[idx_s], msg, sem).wait()
        pltpu.sync_copy(msg, acc.at[idx_d], add=True)
        return carry

    lax.fori_loop(0, G_E, step, 0)


def _cnt_edges(wid, e_hbm, acc, ones_v, idx_d):
    def step(j, carry):
        pltpu.sync_copy(e_hbm.at[1, wid, j], idx_d)
        pltpu.sync_copy(ones_v, acc.at[idx_d], add=True)
        return carry

    lax.fori_loop(0, G_E, step, 0)


def _seg_body_l1(xu_hbm, xm_hbm, eR_hbm, eD_hbm, zs_hbm, ones_hbm,
                 os_m, os_u, oc_m, oc_u,
                 idx_s, idx_d, msg, acc_sum, sem):
    cid = lax.axis_index("c")
    sid = lax.axis_index("s")
    wid = cid * 16 + sid
    r0 = sid * ROWS_PER_TILE

    def zero_acc():
        pltpu.sync_copy(zs_hbm.at[pl.ds(r0, ROWS_PER_TILE)],
                        acc_sum.at[pl.ds(r0, ROWS_PER_TILE)])

    def copy_out(o):
        pltpu.sync_copy(acc_sum.at[pl.ds(r0, ROWS_PER_TILE)],
                        o.at[cid, pl.ds(r0, ROWS_PER_TILE)])

    zero_acc()
    plsc.subcore_barrier()
    _sum_edges(wid, xu_hbm, eR_hbm, acc_sum, idx_s, idx_d, msg, sem)
    plsc.subcore_barrier()
    copy_out(os_m)
    plsc.subcore_barrier()
    zero_acc()
    plsc.subcore_barrier()
    _sum_edges(wid, xm_hbm, eD_hbm, acc_sum, idx_s, idx_d, msg, sem)
    plsc.subcore_barrier()
    copy_out(os_u)
    plsc.subcore_barrier()
    zero_acc()
    pltpu.sync_copy(ones_hbm, msg)
    plsc.subcore_barrier()
    _cnt_edges(wid, eR_hbm, acc_sum, msg, idx_d)
    plsc.subcore_barrier()
    copy_out(oc_m)
    plsc.subcore_barrier()
    zero_acc()
    plsc.subcore_barrier()
    _cnt_edges(wid, eD_hbm, acc_sum, msg, idx_d)
    plsc.subcore_barrier()
    copy_out(oc_u)


def _sc_segsum_l1(xu, xm, eR, eD):
    f32 = jnp.float32
    zs = jnp.zeros((N_PAD, H), f32)
    ones = jnp.ones((GRP, H), f32)
    return pl.kernel(
        _seg_body_l1,
        out_type=[jax.ShapeDtypeStruct((2, N_PAD, H), f32)] * 4,
        mesh=_sc_mesh(),
        scratch_types=[
            pltpu.VMEM((GRP,), jnp.int32),
            pltpu.VMEM((GRP,), jnp.int32),
            pltpu.VMEM((GRP, H), f32),
            pltpu.VMEM_SHARED((N_PAD, H), f32),
            pltpu.SemaphoreType.DMA,
        ],
    )(xu, xm, eR, eD, zs, ones)


def _seg_body_plain(xu_hbm, xm_hbm, eR_hbm, eD_hbm, zs_hbm,
                    os_m, os_u,
                    idx_s, idx_d, msg, acc_sum, sem):
    cid = lax.axis_index("c")
    sid = lax.axis_index("s")
    wid = cid * 16 + sid
    r0 = sid * ROWS_PER_TILE

    def zero_acc():
        pltpu.sync_copy(zs_hbm.at[pl.ds(r0, ROWS_PER_TILE)],
                        acc_sum.at[pl.ds(r0, ROWS_PER_TILE)])

    def copy_out(o):
        pltpu.sync_copy(acc_sum.at[pl.ds(r0, ROWS_PER_TILE)],
                        o.at[cid, pl.ds(r0, ROWS_PER_TILE)])

    zero_acc()
    plsc.subcore_barrier()
    _sum_edges(wid, xu_hbm, eR_hbm, acc_sum, idx_s, idx_d, msg, sem)
    plsc.subcore_barrier()
    copy_out(os_m)
    plsc.subcore_barrier()
    zero_acc()
    plsc.subcore_barrier()
    _sum_edges(wid, xm_hbm, eD_hbm, acc_sum, idx_s, idx_d, msg, sem)
    plsc.subcore_barrier()
    copy_out(os_u)


def _sc_segsum(table_m, table_u, eR, eD):
    f32 = jnp.float32
    zs = jnp.zeros((N_PAD, H), f32)
    return pl.kernel(
        _seg_body_plain,
        out_type=[jax.ShapeDtypeStruct((2, N_PAD, H), f32),
                  jax.ShapeDtypeStruct((2, N_PAD, H), f32)],
        mesh=_sc_mesh(),
        scratch_types=[
            pltpu.VMEM((GRP,), jnp.int32),
            pltpu.VMEM((GRP,), jnp.int32),
            pltpu.VMEM((GRP, H), f32),
            pltpu.VMEM_SHARED((N_PAD, H), f32),
            pltpu.SemaphoreType.DMA,
        ],
    )(table_m, table_u, eR, eD, zs)


def _gather_body(u2_hbm, m2_hbm, eli_hbm, fu_hbm, fm_hbm, idx0, idx1, buf, sem):
    cid = lax.axis_index("c")
    sid = lax.axis_index("s")
    wid = cid * 16 + sid
    base = wid * (G_L * GRP)

    def step(j, carry):
        pltpu.sync_copy(eli_hbm.at[0, wid, j], idx0)
        pltpu.sync_copy(eli_hbm.at[1, wid, j], idx1)
        pltpu.async_copy(u2_hbm.at[idx0], buf, sem).wait()
        pltpu.sync_copy(buf, fu_hbm.at[pl.ds(base + j * GRP, GRP)])
        pltpu.async_copy(m2_hbm.at[idx1], buf, sem).wait()
        pltpu.sync_copy(buf, fm_hbm.at[pl.ds(base + j * GRP, GRP)])
        return carry

    lax.fori_loop(0, G_L, step, 0)


def _sc_label_gather(u2, m2, eli):
    f32 = jnp.float32
    return pl.kernel(
        _gather_body,
        out_type=[jax.ShapeDtypeStruct((L_PAD, H), f32),
                  jax.ShapeDtypeStruct((L_PAD, H), f32)],
        mesh=_sc_mesh(),
        scratch_types=[
            pltpu.VMEM((GRP,), jnp.int32),
            pltpu.VMEM((GRP,), jnp.int32),
            pltpu.VMEM((GRP, H), f32),
            pltpu.SemaphoreType.DMA,
        ],
    )(u2, m2, eli)


def _make_sage_body(relu, out_bf16):
    def body(sm0, sm1, cm0, cm1, xdm, Wlm, blm, Wrm,
             su0, su1, cu0, cu1, xdu, Wlu, blu, Wru, om_ref, ou_ref):
        def one(s0, s1, c0, c1, xd, Wl, bl, Wr, o_ref):
            cnt = jnp.maximum(c0[:, 0:1] + c1[:, 0:1], 1.0)
            mean = (s0[...] + s1[...]) / cnt
            r = _matT(mean, Wl[...]) + bl[...] + _matT(xd[...], Wr[...])
            if relu:
                r = jnp.maximum(r, 0.0)
            o_ref[...] = r.astype(jnp.bfloat16) if out_bf16 else r

        one(sm0, sm1, cm0, cm1, xdm, Wlm, blm, Wrm, om_ref)
        one(su0, su1, cu0, cu1, xdu, Wlu, blu, Wru, ou_ref)

    return body


_SAGE_BLK = 2592


def _tc_sage(relu, out_bf16, sm, cm, xdm, Wlm, blm, Wrm, su, cu, xdu, Wlu, blu, Wru):
    row = pl.BlockSpec((_SAGE_BLK, H), lambda i: (i, 0))
    wfull = pl.BlockSpec((H, H), lambda i: (0, 0))
    bfull = pl.BlockSpec((1, H), lambda i: (0, 0))
    grid = (N_PAD // _SAGE_BLK,)
    odt = jnp.bfloat16 if out_bf16 else jnp.float32
    return pl.pallas_call(
        _make_sage_body(relu, out_bf16),
        grid=grid,
        in_specs=[row, row, row, row, row, wfull, bfull, wfull,
                  row, row, row, row, row, wfull, bfull, wfull],
        out_specs=[row, row],
        out_shape=[jax.ShapeDtypeStruct((N_PAD, H), odt),
                   jax.ShapeDtypeStruct((N_PAD, H), odt)],
    )(sm[0], sm[1], cm[0], cm[1], xdm, Wlm, blm.reshape(1, H), Wrm,
      su[0], su[1], cu[0], cu[1], xdu, Wlu, blu.reshape(1, H), Wru)


_DEC_BLK = 2048


def _dec_body(fu, fm, W1a, W1b, b1, W2, b2, W3, b3, o_ref):
    h = jnp.maximum(_matT(fu[...].astype(jnp.bfloat16), W1a[...])
                    + _matT(fm[...].astype(jnp.bfloat16), W1b[...]) + b1[...], 0.0)
    h = jnp.maximum(_matT(h.astype(jnp.bfloat16), W2[...]) + b2[...], 0.0)
    o_ref[...] = _matT(h.astype(jnp.bfloat16), W3[...]) + b3[...]


def _tc_decoder(fu, fm, d_W1, d_b1, d_W2, d_b2, d_W3, d_b3):
    H4, H2 = 4 * H, 2 * H
    bf16 = jnp.bfloat16
    W1a = d_W1[:, :H].astype(bf16)
    W1b = d_W1[:, H:].astype(bf16)
    W2b = d_W2.astype(bf16)
    W3p = jnp.zeros((8, H2), jnp.float32).at[0].set(d_W3[0]).astype(bf16)
    b3p = jnp.broadcast_to(d_b3.reshape(1, 1), (_DEC_BLK, 8))
    row = pl.BlockSpec((_DEC_BLK, H), lambda i: (i, 0))
    grid = (L_PAD // _DEC_BLK,)
    out = pl.pallas_call(
        _dec_body,
        grid=grid,
        in_specs=[row, row,
                  pl.BlockSpec((H4, H), lambda i: (0, 0)),
                  pl.BlockSpec((H4, H), lambda i: (0, 0)),
                  pl.BlockSpec((1, H4), lambda i: (0, 0)),
                  pl.BlockSpec((H2, H4), lambda i: (0, 0)),
                  pl.BlockSpec((1, H2), lambda i: (0, 0)),
                  pl.BlockSpec((8, H2), lambda i: (0, 0)),
                  pl.BlockSpec((_DEC_BLK, 8), lambda i: (0, 0))],
        out_specs=pl.BlockSpec((_DEC_BLK, 8), lambda i: (i, 0)),
        out_shape=jax.ShapeDtypeStruct((L_PAD, 8), jnp.float32),
    )(fu, fm, W1a, W1b, d_b1.reshape(1, H4), W2b, d_b2.reshape(1, H2),
      W3p, b3p)
    return out[:E_LBL, 0]


def _pad_edges(ei):
    pad = E_PAD - E_EDGE
    trash = N_NODE + jnp.arange(pad, dtype=jnp.int32) % (N_PAD - N_NODE)
    src = jnp.concatenate([ei[0], jnp.zeros((pad,), jnp.int32)])
    dst = jnp.concatenate([ei[1], trash])
    return jnp.stack([src, dst]).reshape(2, NTILE, G_E, GRP)


def kernel(x_user, x_movie, ei_rates, ei_rated, edge_label_index,
           lu_W, lu_b, lm_W, lm_b, bn_u_g, bn_u_b, bn_m_g, bn_m_b,
           c1r_Wl, c1r_bl, c1r_Wr, c1d_Wl, c1d_bl, c1d_Wr,
           c2r_Wl, c2r_bl, c2r_Wr, c2d_Wl, c2d_bl, c2d_Wr,
           d_W1, d_b1, d_W2, d_b2, d_W3, d_b3):
    eR = _pad_edges(ei_rates)
    eD = _pad_edges(ei_rated)
    lpad = L_PAD - E_LBL
    eli = jnp.concatenate([edge_label_index,
                           jnp.zeros((2, lpad), jnp.int32)], axis=1).reshape(2, NTILE, G_L, GRP)

    xu, xm = _tc_input(x_user, x_movie, lu_W, lu_b, lm_W, lm_b,
                       bn_u_g, bn_u_b, bn_m_g, bn_m_b)

    os_m, os_u, oc_m, oc_u = _sc_segsum_l1(xu, xm, eR, eD)

    m1, u1 = _tc_sage(True, False, os_m, oc_m, xm, c1r_Wl, c1r_bl, c1r_Wr,
                      os_u, oc_u, xu, c1d_Wl, c1d_bl, c1d_Wr)

    os2_m, os2_u = _sc_segsum(u1, m1, eR, eD)

    m2, u2 = _tc_sage(False, False, os2_m, oc_m, m1, c2r_Wl, c2r_bl, c2r_Wr,
                      os2_u, oc_u, u1, c2d_Wl, c2d_bl, c2d_Wr)

    fu, fm = _sc_label_gather(u2, m2, eli)

    return _tc_decoder(fu, fm, d_W1, d_b1, d_W2, d_b2, d_W3, d_b3)


# speedup vs baseline: 1.8411x; 1.2757x over previous
import jax
import jax.numpy as jnp
from jax import lax
from jax.experimental import pallas as pl
from jax.experimental.pallas import tpu as pltpu
from jax.experimental.pallas import tpu_sc as plsc

H = 128
N_NODE = 10000
N_PAD = 10368
E_EDGE = 320000
GRP = 128
NTILE = 32
G_E = 79
E_PAD = NTILE * G_E * GRP
E_LBL = 100000
G_L = 25
L_PAD = NTILE * G_L * GRP
ROWS_PER_TILE = N_PAD // 16

_DN = (((1,), (1,)), ((), ()))


def _matT(x, w):
    return lax.dot_general(x, w, _DN, preferred_element_type=jnp.float32)


def _input_body(xu_ref, xm_ref, luW, lub, lmW, lmb, bug, bub, bmg, bmb,
                ou_ref, om_ref):
    def one(x, W, b, g, bb, o_ref):
        y = _matT(x, W[...]) + b[...]
        m = jnp.mean(y, axis=0, keepdims=True)
        v = jnp.mean((y - m) ** 2, axis=0, keepdims=True)
        act = jnp.maximum((y - m) * lax.rsqrt(v + 1e-5) * g[...] + bb[...], 0.0)
        o_ref[...] = jnp.concatenate(
            [act, jnp.zeros((N_PAD - N_NODE, H), jnp.float32)], axis=0)

    one(xu_ref[...], luW, lub, bug, bub, ou_ref)
    one(xm_ref[...], lmW, lmb, bmg, bmb, om_ref)


def _tc_input(x_user, x_movie, lu_W, lu_b, lm_W, lm_b, bn_u_g, bn_u_b, bn_m_g, bn_m_b):
    return pl.pallas_call(
        _input_body,
        out_shape=[jax.ShapeDtypeStruct((N_PAD, H), jnp.float32),
                   jax.ShapeDtypeStruct((N_PAD, H), jnp.float32)],
    )(x_user, x_movie, lu_W, lu_b.reshape(1, H), lm_W, lm_b.reshape(1, H),
      bn_u_g.reshape(1, H), bn_u_b.reshape(1, H), bn_m_g.reshape(1, H), bn_m_b.reshape(1, H))


def _sc_mesh():
    return plsc.VectorSubcoreMesh(core_axis_name="c", subcore_axis_name="s")


def _sum_edges(wid, table_hbm, e_hbm, acc, idxs, idxd, msgs, semg, sems):
    for b in (0, 1):
        pltpu.sync_copy(e_hbm.at[0, wid, b], idxs[b])
        pltpu.sync_copy(e_hbm.at[1, wid, b], idxd[b])
        pltpu.async_copy(table_hbm.at[idxs[b]], msgs[b], semg[b])

    def body(g, carry):
        descs = []
        for b in (0, 1):
            pltpu.make_async_copy(table_hbm.at[idxs[b]], msgs[b], semg[b]).wait()
            descs.append(pltpu.async_copy(msgs[b], acc.at[idxd[b]], sems[b], add=True))
        for b in (0, 1):
            descs[b].wait()
            nx = 2 * g + b + 2

            @pl.when(nx < G_E)
            def _():
                pltpu.sync_copy(e_hbm.at[0, wid, nx], idxs[b])
                pltpu.sync_copy(e_hbm.at[1, wid, nx], idxd[b])
                pltpu.async_copy(table_hbm.at[idxs[b]], msgs[b], semg[b])
        return carry

    lax.fori_loop(0, G_E // 2, body, 0)
    if G_E % 2 == 1:
        pltpu.make_async_copy(table_hbm.at[idxs[0]], msgs[0], semg[0]).wait()
        pltpu.sync_copy(msgs[0], acc.at[idxd[0]], add=True)


def _cnt_edges(wid, e_hbm, acc, ones_v, idxd, sems):
    for b in (0, 1):
        pltpu.sync_copy(e_hbm.at[1, wid, b], idxd[b])

    def body(g, carry):
        descs = []
        for b in (0, 1):
            descs.append(pltpu.async_copy(ones_v, acc.at[idxd[b]], sems[b], add=True))
        for b in (0, 1):
            descs[b].wait()
            nx = 2 * g + b + 2

            @pl.when(nx < G_E)
            def _():
                pltpu.sync_copy(e_hbm.at[1, wid, nx], idxd[b])
        return carry

    lax.fori_loop(0, G_E // 2, body, 0)
    if G_E % 2 == 1:
        pltpu.sync_copy(ones_v, acc.at[idxd[0]], add=True)


def _seg_body_l1(xu_hbm, xm_hbm, eR_hbm, eD_hbm, zs_hbm, ones_hbm,
                 os_m, os_u, oc_m, oc_u,
                 idx_s0, idx_s1, idx_d0, idx_d1, msg0, msg1, acc_sum,
                 sem_g0, sem_g1, sem_s0, sem_s1):
    cid = lax.axis_index("c")
    sid = lax.axis_index("s")
    wid = cid * 16 + sid
    r0 = sid * ROWS_PER_TILE
    idxs, idxd = (idx_s0, idx_s1), (idx_d0, idx_d1)
    msgs = (msg0, msg1)
    semg, sems = (sem_g0, sem_g1), (sem_s0, sem_s1)

    def zero_acc():
        pltpu.sync_copy(zs_hbm.at[pl.ds(r0, ROWS_PER_TILE)],
                        acc_sum.at[pl.ds(r0, ROWS_PER_TILE)])

    def copy_out(o):
        pltpu.sync_copy(acc_sum.at[pl.ds(r0, ROWS_PER_TILE)],
                        o.at[cid, pl.ds(r0, ROWS_PER_TILE)])

    zero_acc()
    plsc.subcore_barrier()
    _sum_edges(wid, xu_hbm, eR_hbm, acc_sum, idxs, idxd, msgs, semg, sems)
    plsc.subcore_barrier()
    copy_out(os_m)
    plsc.subcore_barrier()
    zero_acc()
    plsc.subcore_barrier()
    _sum_edges(wid, xm_hbm, eD_hbm, acc_sum, idxs, idxd, msgs, semg, sems)
    plsc.subcore_barrier()
    copy_out(os_u)
    plsc.subcore_barrier()
    zero_acc()
    pltpu.sync_copy(ones_hbm, msg0)
    plsc.subcore_barrier()
    _cnt_edges(wid, eR_hbm, acc_sum, msg0, idxd, sems)
    plsc.subcore_barrier()
    copy_out(oc_m)
    plsc.subcore_barrier()
    zero_acc()
    plsc.subcore_barrier()
    _cnt_edges(wid, eD_hbm, acc_sum, msg0, idxd, sems)
    plsc.subcore_barrier()
    copy_out(oc_u)


def _sc_segsum_l1(xu, xm, eR, eD):
    f32 = jnp.float32
    zs = jnp.zeros((N_PAD, H), f32)
    ones = jnp.ones((GRP, H), f32)
    return pl.kernel(
        _seg_body_l1,
        out_type=[jax.ShapeDtypeStruct((2, N_PAD, H), f32)] * 4,
        mesh=_sc_mesh(),
        scratch_types=[
            pltpu.VMEM((GRP,), jnp.int32),
            pltpu.VMEM((GRP,), jnp.int32),
            pltpu.VMEM((GRP,), jnp.int32),
            pltpu.VMEM((GRP,), jnp.int32),
            pltpu.VMEM((GRP, H), f32),
            pltpu.VMEM((GRP, H), f32),
            pltpu.VMEM_SHARED((N_PAD, H), f32),
            pltpu.SemaphoreType.DMA,
            pltpu.SemaphoreType.DMA,
            pltpu.SemaphoreType.DMA,
            pltpu.SemaphoreType.DMA,
        ],
    )(xu, xm, eR, eD, zs, ones)


def _seg_body_plain(xu_hbm, xm_hbm, eR_hbm, eD_hbm, zs_hbm,
                    os_m, os_u,
                    idx_s0, idx_s1, idx_d0, idx_d1, msg0, msg1, acc_sum,
                    sem_g0, sem_g1, sem_s0, sem_s1):
    cid = lax.axis_index("c")
    sid = lax.axis_index("s")
    wid = cid * 16 + sid
    r0 = sid * ROWS_PER_TILE
    idxs, idxd = (idx_s0, idx_s1), (idx_d0, idx_d1)
    msgs = (msg0, msg1)
    semg, sems = (sem_g0, sem_g1), (sem_s0, sem_s1)

    def zero_acc():
        pltpu.sync_copy(zs_hbm.at[pl.ds(r0, ROWS_PER_TILE)],
                        acc_sum.at[pl.ds(r0, ROWS_PER_TILE)])

    def copy_out(o):
        pltpu.sync_copy(acc_sum.at[pl.ds(r0, ROWS_PER_TILE)],
                        o.at[cid, pl.ds(r0, ROWS_PER_TILE)])

    zero_acc()
    plsc.subcore_barrier()
    _sum_edges(wid, xu_hbm, eR_hbm, acc_sum, idxs, idxd, msgs, semg, sems)
    plsc.subcore_barrier()
    copy_out(os_m)
    plsc.subcore_barrier()
    zero_acc()
    plsc.subcore_barrier()
    _sum_edges(wid, xm_hbm, eD_hbm, acc_sum, idxs, idxd, msgs, semg, sems)
    plsc.subcore_barrier()
    copy_out(os_u)


def _sc_segsum(table_m, table_u, eR, eD):
    f32 = jnp.float32
    zs = jnp.zeros((N_PAD, H), f32)
    return pl.kernel(
        _seg_body_plain,
        out_type=[jax.ShapeDtypeStruct((2, N_PAD, H), f32),
                  jax.ShapeDtypeStruct((2, N_PAD, H), f32)],
        mesh=_sc_mesh(),
        scratch_types=[
            pltpu.VMEM((GRP,), jnp.int32),
            pltpu.VMEM((GRP,), jnp.int32),
            pltpu.VMEM((GRP,), jnp.int32),
            pltpu.VMEM((GRP,), jnp.int32),
            pltpu.VMEM((GRP, H), f32),
            pltpu.VMEM((GRP, H), f32),
            pltpu.VMEM_SHARED((N_PAD, H), f32),
            pltpu.SemaphoreType.DMA,
            pltpu.SemaphoreType.DMA,
            pltpu.SemaphoreType.DMA,
            pltpu.SemaphoreType.DMA,
        ],
    )(table_m, table_u, eR, eD, zs)


def _gather_body(u2_hbm, m2_hbm, eli_hbm, fu_hbm, fm_hbm,
                 idx0a, idx0b, idx1a, idx1b, bufU0, bufU1, bufM0, bufM1,
                 sem_gu0, sem_gu1, sem_gm0, sem_gm1, sem_w0, sem_w1):
    cid = lax.axis_index("c")
    sid = lax.axis_index("s")
    wid = cid * 16 + sid
    base = wid * (G_L * GRP)
    idx0, idx1 = (idx0a, idx0b), (idx1a, idx1b)
    bufU, bufM = (bufU0, bufU1), (bufM0, bufM1)
    semgu, semgm = (sem_gu0, sem_gu1), (sem_gm0, sem_gm1)
    semw = (sem_w0, sem_w1)

    for b in (0, 1):
        pltpu.sync_copy(eli_hbm.at[0, wid, b], idx0[b])
        pltpu.sync_copy(eli_hbm.at[1, wid, b], idx1[b])
        pltpu.async_copy(u2_hbm.at[idx0[b]], bufU[b], semgu[b])
        pltpu.async_copy(m2_hbm.at[idx1[b]], bufM[b], semgm[b])

    def body(g, carry):
        descs = []
        for b in (0, 1):
            j = 2 * g + b
            pltpu.make_async_copy(u2_hbm.at[idx0[b]], bufU[b], semgu[b]).wait()
            du = pltpu.async_copy(bufU[b], fu_hbm.at[pl.ds(base + j * GRP, GRP)], semw[b])
            pltpu.make_async_copy(m2_hbm.at[idx1[b]], bufM[b], semgm[b]).wait()
            dm = pltpu.async_copy(bufM[b], fm_hbm.at[pl.ds(base + j * GRP, GRP)], semw[b])
            descs.append((du, dm))
        for b in (0, 1):
            descs[b][0].wait()
            descs[b][1].wait()
            nx = 2 * g + b + 2

            @pl.when(nx < G_L)
            def _():
                pltpu.sync_copy(eli_hbm.at[0, wid, nx], idx0[b])
                pltpu.sync_copy(eli_hbm.at[1, wid, nx], idx1[b])
                pltpu.async_copy(u2_hbm.at[idx0[b]], bufU[b], semgu[b])
                pltpu.async_copy(m2_hbm.at[idx1[b]], bufM[b], semgm[b])
        return carry

    lax.fori_loop(0, G_L // 2, body, 0)
    if G_L % 2 == 1:
        jl = G_L - 1
        pltpu.make_async_copy(u2_hbm.at[idx0[0]], bufU[0], semgu[0]).wait()
        pltpu.sync_copy(bufU[0], fu_hbm.at[pl.ds(base + jl * GRP, GRP)])
        pltpu.make_async_copy(m2_hbm.at[idx1[0]], bufM[0], semgm[0]).wait()
        pltpu.sync_copy(bufM[0], fm_hbm.at[pl.ds(base + jl * GRP, GRP)])


def _sc_label_gather(u2, m2, eli):
    f32 = jnp.float32
    return pl.kernel(
        _gather_body,
        out_type=[jax.ShapeDtypeStruct((L_PAD, H), f32),
                  jax.ShapeDtypeStruct((L_PAD, H), f32)],
        mesh=_sc_mesh(),
        scratch_types=[
            pltpu.VMEM((GRP,), jnp.int32),
            pltpu.VMEM((GRP,), jnp.int32),
            pltpu.VMEM((GRP,), jnp.int32),
            pltpu.VMEM((GRP,), jnp.int32),
            pltpu.VMEM((GRP, H), f32),
            pltpu.VMEM((GRP, H), f32),
            pltpu.VMEM((GRP, H), f32),
            pltpu.VMEM((GRP, H), f32),
            pltpu.SemaphoreType.DMA,
            pltpu.SemaphoreType.DMA,
            pltpu.SemaphoreType.DMA,
            pltpu.SemaphoreType.DMA,
            pltpu.SemaphoreType.DMA,
            pltpu.SemaphoreType.DMA,
        ],
    )(u2, m2, eli)


def _make_sage_body(relu, out_bf16):
    def body(sm0, sm1, cm0, cm1, xdm, Wlm, blm, Wrm,
             su0, su1, cu0, cu1, xdu, Wlu, blu, Wru, om_ref, ou_ref):
        def one(s0, s1, c0, c1, xd, Wl, bl, Wr, o_ref):
            cnt = jnp.maximum(c0[:, 0:1] + c1[:, 0:1], 1.0)
            mean = (s0[...] + s1[...]) / cnt
            r = _matT(mean, Wl[...]) + bl[...] + _matT(xd[...], Wr[...])
            if relu:
                r = jnp.maximum(r, 0.0)
            o_ref[...] = r.astype(jnp.bfloat16) if out_bf16 else r

        one(sm0, sm1, cm0, cm1, xdm, Wlm, blm, Wrm, om_ref)
        one(su0, su1, cu0, cu1, xdu, Wlu, blu, Wru, ou_ref)

    return body


_SAGE_BLK = 2592


def _tc_sage(relu, out_bf16, sm, cm, xdm, Wlm, blm, Wrm, su, cu, xdu, Wlu, blu, Wru):
    row = pl.BlockSpec((_SAGE_BLK, H), lambda i: (i, 0))
    wfull = pl.BlockSpec((H, H), lambda i: (0, 0))
    bfull = pl.BlockSpec((1, H), lambda i: (0, 0))
    grid = (N_PAD // _SAGE_BLK,)
    odt = jnp.bfloat16 if out_bf16 else jnp.float32
    return pl.pallas_call(
        _make_sage_body(relu, out_bf16),
        grid=grid,
        in_specs=[row, row, row, row, row, wfull, bfull, wfull,
                  row, row, row, row, row, wfull, bfull, wfull],
        out_specs=[row, row],
        out_shape=[jax.ShapeDtypeStruct((N_PAD, H), odt),
                   jax.ShapeDtypeStruct((N_PAD, H), odt)],
    )(sm[0], sm[1], cm[0], cm[1], xdm, Wlm, blm.reshape(1, H), Wrm,
      su[0], su[1], cu[0], cu[1], xdu, Wlu, blu.reshape(1, H), Wru)


_DEC_BLK = 2048


def _dec_body(fu, fm, W1a, W1b, b1, W2, b2, W3, b3, o_ref):
    h = jnp.maximum(_matT(fu[...].astype(jnp.bfloat16), W1a[...])
                    + _matT(fm[...].astype(jnp.bfloat16), W1b[...]) + b1[...], 0.0)
    h = jnp.maximum(_matT(h.astype(jnp.bfloat16), W2[...]) + b2[...], 0.0)
    o_ref[...] = _matT(h.astype(jnp.bfloat16), W3[...]) + b3[...]


def _tc_decoder(fu, fm, d_W1, d_b1, d_W2, d_b2, d_W3, d_b3):
    H4, H2 = 4 * H, 2 * H
    bf16 = jnp.bfloat16
    W1a = d_W1[:, :H].astype(bf16)
    W1b = d_W1[:, H:].astype(bf16)
    W2b = d_W2.astype(bf16)
    W3p = jnp.zeros((8, H2), jnp.float32).at[0].set(d_W3[0]).astype(bf16)
    b3p = jnp.broadcast_to(d_b3.reshape(1, 1), (_DEC_BLK, 8))
    row = pl.BlockSpec((_DEC_BLK, H), lambda i: (i, 0))
    grid = (L_PAD // _DEC_BLK,)
    out = pl.pallas_call(
        _dec_body,
        grid=grid,
        in_specs=[row, row,
                  pl.BlockSpec((H4, H), lambda i: (0, 0)),
                  pl.BlockSpec((H4, H), lambda i: (0, 0)),
                  pl.BlockSpec((1, H4), lambda i: (0, 0)),
                  pl.BlockSpec((H2, H4), lambda i: (0, 0)),
                  pl.BlockSpec((1, H2), lambda i: (0, 0)),
                  pl.BlockSpec((8, H2), lambda i: (0, 0)),
                  pl.BlockSpec((_DEC_BLK, 8), lambda i: (0, 0))],
        out_specs=pl.BlockSpec((_DEC_BLK, 8), lambda i: (i, 0)),
        out_shape=jax.ShapeDtypeStruct((L_PAD, 8), jnp.float32),
    )(fu, fm, W1a, W1b, d_b1.reshape(1, H4), W2b, d_b2.reshape(1, H2),
      W3p, b3p)
    return out[:E_LBL, 0]


def _pad_edges(ei):
    pad = E_PAD - E_EDGE
    trash = N_NODE + jnp.arange(pad, dtype=jnp.int32) % (N_PAD - N_NODE)
    src = jnp.concatenate([ei[0], jnp.zeros((pad,), jnp.int32)])
    dst = jnp.concatenate([ei[1], trash])
    return jnp.stack([src, dst]).reshape(2, NTILE, G_E, GRP)


def kernel(x_user, x_movie, ei_rates, ei_rated, edge_label_index,
           lu_W, lu_b, lm_W, lm_b, bn_u_g, bn_u_b, bn_m_g, bn_m_b,
           c1r_Wl, c1r_bl, c1r_Wr, c1d_Wl, c1d_bl, c1d_Wr,
           c2r_Wl, c2r_bl, c2r_Wr, c2d_Wl, c2d_bl, c2d_Wr,
           d_W1, d_b1, d_W2, d_b2, d_W3, d_b3):
    eR = _pad_edges(ei_rates)
    eD = _pad_edges(ei_rated)
    lpad = L_PAD - E_LBL
    eli = jnp.concatenate([edge_label_index,
                           jnp.zeros((2, lpad), jnp.int32)], axis=1).reshape(2, NTILE, G_L, GRP)

    xu, xm = _tc_input(x_user, x_movie, lu_W, lu_b, lm_W, lm_b,
                       bn_u_g, bn_u_b, bn_m_g, bn_m_b)

    os_m, os_u, oc_m, oc_u = _sc_segsum_l1(xu, xm, eR, eD)

    m1, u1 = _tc_sage(True, False, os_m, oc_m, xm, c1r_Wl, c1r_bl, c1r_Wr,
                      os_u, oc_u, xu, c1d_Wl, c1d_bl, c1d_Wr)

    os2_m, os2_u = _sc_segsum(u1, m1, eR, eD)

    m2, u2 = _tc_sage(False, False, os2_m, oc_m, m1, c2r_Wl, c2r_bl, c2r_Wr,
                      os2_u, oc_u, u1, c2d_Wl, c2d_bl, c2d_Wr)

    fu, fm = _sc_label_gather(u2, m2, eli)

    return _tc_decoder(fu, fm, d_W1, d_b1, d_W2, d_b2, d_W3, d_b3)
